# baseline (speedup 1.0000x reference)
import functools

import jax
import jax.numpy as jnp
from jax import lax
from jax.experimental import pallas as pl
from jax.experimental.pallas import tpu as pltpu
from jax.experimental.pallas import tpu_sc as plsc

E_DIM = 32
NUM_ASP = 18
LANE = 128
RPG = LANE // E_DIM
CHUNK = 128


@functools.cache
def _make_gather(B: int, n_groups: int, n_rows: int):
    info = plsc.get_sparse_core_info()
    NC, NS = info.num_cores, info.num_subcores
    NW = NC * NS
    assert B % (8 * NW) == 0
    bpw = B // NW
    nch = bpw // CHUNK
    assert bpw % CHUNK == 0

    mesh = plsc.VectorSubcoreMesh(core_axis_name="c", subcore_axis_name="s")
    f32 = jnp.float32
    i32 = jnp.int32

    @functools.partial(
        pl.kernel,
        mesh=mesh,
        compiler_params=pltpu.CompilerParams(use_tc_tiling_on_sc=False),
        out_type=(
            jax.ShapeDtypeStruct((B, LANE), f32),
            jax.ShapeDtypeStruct((B, LANE), f32),
            jax.ShapeDtypeStruct((B,), f32),
            jax.ShapeDtypeStruct((B,), f32),
        ),
        scratch_types=[
            pltpu.VMEM((bpw,), i32),
            pltpu.VMEM((bpw,), i32),
            pltpu.VMEM((nch, CHUNK), i32),
            pltpu.VMEM((nch, CHUNK), i32),
            pltpu.VMEM((2, CHUNK, LANE), f32),
            pltpu.VMEM((2, CHUNK, LANE), f32),
            pltpu.VMEM((bpw,), f32),
            pltpu.VMEM((bpw,), f32),
            pltpu.SemaphoreType.DMA,
            pltpu.SemaphoreType.DMA,
            pltpu.SemaphoreType.DMA,
            pltpu.SemaphoreType.DMA,
            pltpu.SemaphoreType.DMA,
            pltpu.SemaphoreType.DMA,
        ],
    )
    def gather(x_hbm, y_hbm, u2_hbm, i2_hbm, ubt_hbm, ibt_hbm,
               u128_out, i128_out, ub_out, ib_out,
               xv, yv, qx, qy, ubuf, ibuf, ubv, ibv,
               su0, su1, si0, si1, sb0, sb1):
        wid = lax.axis_index("s") * NC + lax.axis_index("c")
        base = wid * bpw
        pltpu.sync_copy(x_hbm.at[pl.ds(base, bpw)], xv)
        pltpu.sync_copy(y_hbm.at[pl.ds(base, bpw)], yv)
        for c in range(nch):
            for k in range(CHUNK // 16):
                sl = pl.ds(c * CHUNK + k * 16, 16)
                qx[c, pl.ds(k * 16, 16)] = xv[sl] >> 2
                qy[c, pl.ds(k * 16, 16)] = yv[sl] >> 2
        cb0 = pltpu.async_copy(ubt_hbm.at[xv], ubv, sb0)
        cb1 = pltpu.async_copy(ibt_hbm.at[yv], ibv, sb1)
        usem = (su0, su1)
        isem = (si0, si1)
        pend = {}
        for c in range(nch):
            pend[c] = (
                pltpu.async_copy(u2_hbm.at[qx.at[c]], ubuf.at[c % 2], usem[c % 2]),
                pltpu.async_copy(i2_hbm.at[qy.at[c]], ibuf.at[c % 2], isem[c % 2]),
            )
            if c >= 1:
                cu, ci = pend.pop(c - 1)
                off = base + (c - 1) * CHUNK
                cu.wait()
                pltpu.sync_copy(ubuf.at[(c - 1) % 2], u128_out.at[pl.ds(off, CHUNK)])
                ci.wait()
                pltpu.sync_copy(ibuf.at[(c - 1) % 2], i128_out.at[pl.ds(off, CHUNK)])
        cu, ci = pend.pop(nch - 1)
        off = base + (nch - 1) * CHUNK
        cu.wait()
        pltpu.sync_copy(ubuf.at[(nch - 1) % 2], u128_out.at[pl.ds(off, CHUNK)])
        ci.wait()
        pltpu.sync_copy(ibuf.at[(nch - 1) % 2], i128_out.at[pl.ds(off, CHUNK)])
        cb0.wait()
        pltpu.sync_copy(ubv, ub_out.at[pl.ds(base, bpw)])
        cb1.wait()
        pltpu.sync_copy(ibv, ib_out.at[pl.ds(base, bpw)])

    return gather


def _dot_t(a, b):
    return lax.dot_general(a, b, (((1,), (1,)), ((), ())),
                           preferred_element_type=jnp.float32)


def _extract(rows128, rem):
    r = rows128[:, 0:E_DIM]
    for k in range(1, RPG):
        r = jnp.where(rem == k, rows128[:, k * E_DIM:(k + 1) * E_DIM], r)
    return r


def _dense_body(u128_ref, i128_ref, x_ref, y_ref, ub_ref, ib_ref,
                asp_ref, aw_ref,
                w1_ref, b1_ref, w2_ref, b2_ref, w3_ref, b3_ref,
                out_ref, sim_ref, pref_ref):
    u = _extract(u128_ref[...], x_ref[...] & (RPG - 1))
    it = _extract(i128_ref[...], y_ref[...] & (RPG - 1))
    aw = aw_ref[...]
    out_ref[...] = (jnp.sum(u * it, axis=-1, keepdims=True)
                    + ub_ref[...] + ib_ref[...] + 3.53)
    wa = jnp.sqrt(jnp.sum(aw * aw, axis=1))
    aspv = asp_ref[...]
    t = aspv / jnp.maximum(jnp.abs(aspv) * wa[None, :], 1e-12)
    h = _dot_t(it, w1_ref[...]) + b1_ref[...]
    h = _dot_t(h, w2_ref[...]) + b2_ref[...]
    logits = _dot_t(h, w3_ref[...]) + b3_ref[...]
    weight = 1.0 / (1.0 + jnp.exp(-logits))
    item_asp = lax.dot_general(t * weight, aw, (((1,), (0,)), ((), ())),
                               preferred_element_type=jnp.float32)
    d = item_asp - it + 1e-6
    sim_ref[...] = jnp.sqrt(jnp.sum(d * d, axis=-1, keepdims=True))
    pref_ref[...] = t * _dot_t(u, aw)


def _dense(u128, i128, x2, y2, ub, ib, asp, asp_W, W1, b1, W2, b2, W3, b3):
    B = u128.shape[0]
    BLK = 2048
    grid = (B // BLK,)
    f32 = jnp.float32
    row = lambda b: (b, 0)
    rep = lambda b: (0, 0)
    return pl.pallas_call(
        _dense_body,
        grid=grid,
        in_specs=[
            pl.BlockSpec((BLK, LANE), row),
            pl.BlockSpec((BLK, LANE), row),
            pl.BlockSpec((BLK, 1), row),
            pl.BlockSpec((BLK, 1), row),
            pl.BlockSpec((BLK, 1), row),
            pl.BlockSpec((BLK, 1), row),
            pl.BlockSpec((BLK, NUM_ASP), row),
            pl.BlockSpec((NUM_ASP, E_DIM), rep),
            pl.BlockSpec((50, E_DIM), rep),
            pl.BlockSpec((1, 50), rep),
            pl.BlockSpec((25, 50), rep),
            pl.BlockSpec((1, 25), rep),
            pl.BlockSpec((NUM_ASP, 25), rep),
            pl.BlockSpec((1, NUM_ASP), rep),
        ],
        out_specs=[
            pl.BlockSpec((BLK, 1), row),
            pl.BlockSpec((BLK, 1), row),
            pl.BlockSpec((BLK, NUM_ASP), row),
        ],
        out_shape=[
            jax.ShapeDtypeStruct((B, 1), f32),
            jax.ShapeDtypeStruct((B, 1), f32),
            jax.ShapeDtypeStruct((B, NUM_ASP), f32),
        ],
    )(u128, i128, x2, y2, ub, ib, asp, asp_W,
      W1, b1.reshape(1, 50), W2, b2.reshape(1, 25), W3, b3.reshape(1, NUM_ASP))


def kernel(x, y, asp, user_table, item_table, u_bias, i_bias, asp_W,
           W1, b1, W2, b2, W3, b3):
    B = x.shape[0]
    x = x.astype(jnp.int32)
    y = y.astype(jnp.int32)
    n_rows = user_table.shape[0]
    u2 = user_table.reshape(n_rows // RPG, LANE)
    i2 = item_table.reshape(n_rows // RPG, LANE)
    u128, i128, ub, ib = _make_gather(B, n_rows // RPG, n_rows)(
        x, y, u2, i2, u_bias, i_bias)
    out2, sim2, pref = _dense(u128, i128, x.reshape(B, 1), y.reshape(B, 1),
                              ub.reshape(B, 1), ib.reshape(B, 1), asp, asp_W,
                              W1, b1, W2, b2, W3, b3)
    return out2.reshape(B), sim2.reshape(B), pref


# speedup vs baseline: 3.8348x; 3.8348x over previous
import functools

import jax
import jax.numpy as jnp
from jax import lax
from jax.experimental import pallas as pl
from jax.experimental.pallas import tpu as pltpu
from jax.experimental.pallas import tpu_sc as plsc

E_DIM = 32
NUM_ASP = 18
LANE = 128
CH = 16


@functools.cache
def _make_gather(B: int, n_cols: int):
    info = plsc.get_sparse_core_info()
    NC, NS = info.num_cores, info.num_subcores
    NW = NC * NS
    assert B % (8 * NW) == 0
    bpw = B // NW
    nch = bpw // CH

    mesh = plsc.VectorSubcoreMesh(core_axis_name="c", subcore_axis_name="s")
    f32 = jnp.float32
    i32 = jnp.int32

    @functools.partial(
        pl.kernel,
        mesh=mesh,
        compiler_params=pltpu.CompilerParams(
            use_tc_tiling_on_sc=True, needs_layout_passes=False),
        out_type=(
            jax.ShapeDtypeStruct((E_DIM, B), f32),
            jax.ShapeDtypeStruct((E_DIM, B), f32),
            jax.ShapeDtypeStruct((1, B), f32),
            jax.ShapeDtypeStruct((1, B), f32),
        ),
        scratch_types=[
            pltpu.VMEM((bpw,), i32),
            pltpu.VMEM((bpw,), i32),
            pltpu.VMEM((CH, E_DIM, LANE), f32),
            pltpu.VMEM((E_DIM, bpw), f32),
            pltpu.VMEM((E_DIM, bpw), f32),
            pltpu.VMEM((bpw,), f32),
            pltpu.VMEM((bpw,), f32),
            pltpu.SemaphoreType.DMA,
            pltpu.SemaphoreType.DMA,
            pltpu.SemaphoreType.DMA,
        ],
    )
    def gather(x_hbm, y_hbm, ut_hbm, it_hbm, ubt_hbm, ibt_hbm,
               uT_out, iT_out, ub_out, ib_out,
               xv, yv, sbuf, ucols, icols, ubv, ibv, sd, sb0, sb1):
        wid = lax.axis_index("s") * NC + lax.axis_index("c")
        base = wid * bpw
        pltpu.sync_copy(x_hbm.at[pl.ds(base, bpw)], xv)
        pltpu.sync_copy(y_hbm.at[pl.ds(base, bpw)], yv)
        cb0 = pltpu.async_copy(ubt_hbm.at[xv], ubv, sb0)
        cb1 = pltpu.async_copy(ibt_hbm.at[yv], ibv, sb1)
        iota16 = lax.iota(i32, 16)

        def fetch_one(tab_hbm, idx_vmem, j0):
            xc = plsc.load_gather(idx_vmem, [j0 + iota16])
            qc = xc >> 7
            for k in range(CH):
                qk = lax.reduce_max(
                    jnp.where(iota16 == k, qc, 0), axes=(0,))
                a = pl.multiple_of(qk * LANE, LANE)
                pltpu.async_copy(tab_hbm.at[:, pl.ds(a, LANE)],
                                 sbuf.at[k], sd)
            for k in range(CH):
                pltpu.make_async_copy(
                    tab_hbm.at[:, pl.ds(0, LANE)], sbuf.at[k], sd).wait()
            return xc & (LANE - 1)

        def extract_one(cols, lv, j0):
            col_i = j0 + iota16
            for e in range(E_DIM):
                ev = jnp.full((16,), e, i32)
                v = plsc.load_gather(sbuf, [iota16, ev, lv])
                plsc.store_scatter(cols, [ev, col_i], v)

        def chunk(c, carry):
            j0 = c * CH
            lv = fetch_one(ut_hbm, xv, j0)
            extract_one(ucols, lv, j0)
            lv = fetch_one(it_hbm, yv, j0)
            extract_one(icols, lv, j0)
            return carry

        lax.fori_loop(0, nch, chunk, 0)
        pltpu.sync_copy(ucols, uT_out.at[:, pl.ds(base, bpw)])
        pltpu.sync_copy(icols, iT_out.at[:, pl.ds(base, bpw)])
        cb0.wait()
        pltpu.sync_copy(ubv, ub_out.at[0, pl.ds(base, bpw)])
        cb1.wait()
        pltpu.sync_copy(ibv, ib_out.at[0, pl.ds(base, bpw)])

    return gather


def _mm(a, b):
    return lax.dot_general(a, b, (((1,), (0,)), ((), ())),
                           preferred_element_type=jnp.float32)


def _dense_body(uT_ref, iT_ref, ub_ref, ib_ref, aspT_ref, aw_ref,
                w1_ref, b1_ref, w2_ref, b2_ref, w3_ref, b3_ref,
                out_ref, sim_ref, prefT_ref):
    u = uT_ref[...]
    it = iT_ref[...]
    aw = aw_ref[...]
    out_ref[...] = (jnp.sum(u * it, axis=0, keepdims=True)
                    + ub_ref[...] + ib_ref[...] + 3.53)
    wa = jnp.sqrt(jnp.sum(aw * aw, axis=1, keepdims=True))
    a = aspT_ref[...]
    t = a / jnp.maximum(jnp.abs(a) * wa, 1e-12)
    h = _mm(w1_ref[...], it) + b1_ref[...]
    h = _mm(w2_ref[...], h) + b2_ref[...]
    logits = _mm(w3_ref[...], h) + b3_ref[...]
    weight = 1.0 / (1.0 + jnp.exp(-logits))
    tw = t * weight
    item_asp = lax.dot_general(aw, tw, (((0,), (0,)), ((), ())),
                               preferred_element_type=jnp.float32)
    d = item_asp - it + 1e-6
    sim_ref[...] = jnp.sqrt(jnp.sum(d * d, axis=0, keepdims=True))
    prefT_ref[...] = t * _mm(aw, u)


def _dense(uT, iT, ub, ib, aspT, asp_W, W1, b1, W2, b2, W3, b3):
    B = uT.shape[1]
    BLK = 2048
    grid = (B // BLK,)
    f32 = jnp.float32
    col = lambda b: (0, b)
    rep = lambda b: (0, 0)
    return pl.pallas_call(
        _dense_body,
        grid=grid,
        in_specs=[
            pl.BlockSpec((E_DIM, BLK), col),
            pl.BlockSpec((E_DIM, BLK), col),
            pl.BlockSpec((1, BLK), col),
            pl.BlockSpec((1, BLK), col),
            pl.BlockSpec((NUM_ASP, BLK), col),
            pl.BlockSpec((NUM_ASP, E_DIM), rep),
            pl.BlockSpec((50, E_DIM), rep),
            pl.BlockSpec((50, 1), rep),
            pl.BlockSpec((25, 50), rep),
            pl.BlockSpec((25, 1), rep),
            pl.BlockSpec((NUM_ASP, 25), rep),
            pl.BlockSpec((NUM_ASP, 1), rep),
        ],
        out_specs=[
            pl.BlockSpec((1, BLK), col),
            pl.BlockSpec((1, BLK), col),
            pl.BlockSpec((NUM_ASP, BLK), col),
        ],
        out_shape=[
            jax.ShapeDtypeStruct((1, B), f32),
            jax.ShapeDtypeStruct((1, B), f32),
            jax.ShapeDtypeStruct((NUM_ASP, B), f32),
        ],
    )(uT, iT, ub, ib, aspT, asp_W,
      W1, b1.reshape(50, 1), W2, b2.reshape(25, 1), W3, b3.reshape(NUM_ASP, 1))


def kernel(x, y, asp, user_table, item_table, u_bias, i_bias, asp_W,
           W1, b1, W2, b2, W3, b3):
    B = x.shape[0]
    x = x.astype(jnp.int32)
    y = y.astype(jnp.int32)
    uT, iT, ub, ib = _make_gather(B, user_table.shape[0])(
        x, y, user_table.T, item_table.T, u_bias, i_bias)
    out2, sim2, prefT = _dense(uT, iT, ub, ib, asp.T, asp_W,
                               W1, b1, W2, b2, W3, b3)
    return out2.reshape(B), sim2.reshape(B), prefT.T
